# baseline (speedup 1.0000x reference)
import functools
import math

import jax
import jax.numpy as jnp
import numpy as np
from jax import lax
from jax.experimental import pallas as pl
from jax.experimental.pallas import tpu as pltpu
from jax.experimental.pallas import tpu_sc as plsc

VOCAB = 1000000
DIM = 64
B = 4096
L = 50
MAX_LEN = 5000

NUM_CORES = 2
NUM_SUBCORES = 16
NUM_WORKERS = NUM_CORES * NUM_SUBCORES
LANES = 16
VECS = DIM // LANES

PAD_COLS = 2 * DIM

ROWS = B * L
ROWS_PER_WORKER = ROWS // NUM_WORKERS
SEQ_PER_CHUNK = 4
CHUNK = SEQ_PER_CHUNK * L
NUM_CHUNKS = ROWS_PER_WORKER // CHUNK
NBUF = 2

SCALE = math.sqrt(DIM)


def _build_pe(dim: int, max_len: int) -> np.ndarray:
    position = np.arange(max_len, dtype=np.float32)[:, None]
    div_term = np.exp(
        np.arange(0, dim, 2, dtype=np.float32) * -(math.log(10000.0) / dim)
    )[None, :]
    pe = np.zeros((max_len, dim), dtype=np.float32)
    pe[:, 0::2] = np.sin(position * div_term)
    pe[:, 1::2] = np.cos(position * div_term)
    return pe


_PE_L = _build_pe(DIM, MAX_LEN)[:L]


_MESH = plsc.VectorSubcoreMesh(
    core_axis_name="c", subcore_axis_name="s",
    num_cores=NUM_CORES, num_subcores=NUM_SUBCORES,
)


@functools.partial(
    pl.kernel,
    out_type=jax.ShapeDtypeStruct((ROWS, DIM), jnp.float32),
    mesh=_MESH,
    scratch_types=[
        pltpu.VMEM((ROWS_PER_WORKER,), jnp.int32),
        pltpu.VMEM((L, DIM), jnp.float32),
        [pltpu.VMEM((CHUNK, PAD_COLS), jnp.float32) for _ in range(NBUF)],
        [pltpu.VMEM((CHUNK, DIM), jnp.float32) for _ in range(NBUF)],
        [pltpu.VMEM((CHUNK,), jnp.int32) for _ in range(NBUF)],
        [pltpu.SemaphoreType.DMA for _ in range(NBUF)],
        [pltpu.SemaphoreType.DMA for _ in range(NBUF)],
    ],
    compiler_params=pltpu.CompilerParams(use_tc_tiling_on_sc=True),
)
def _embed_sc(ids_hbm, table_hbm, bias_hbm, out_hbm, idx_v, bias_v, bufs,
              obufs, cidxs, gsems, ssems):
    wid = lax.axis_index("s") * NUM_CORES + lax.axis_index("c")
    base = wid * ROWS_PER_WORKER

    pltpu.sync_copy(bias_hbm, bias_v)
    pltpu.sync_copy(ids_hbm.at[pl.ds(base, ROWS_PER_WORKER)], idx_v)

    def prep(g):
        q = g % NBUF
        starts = list(range(0, CHUNK - LANES + 1, LANES))
        if starts[-1] + LANES < CHUNK:
            starts.append(CHUNK - LANES)
        for j in starts:
            cidxs[q][pl.ds(j, LANES)] = idx_v[pl.ds(g * CHUNK + j, LANES)]

    def start_gather(g):
        q = g % NBUF
        return pltpu.async_copy(table_hbm.at[cidxs[q]], bufs[q], gsems[q])

    def compute(p):
        buf, obuf = bufs[p], obufs[p]

        def pos_body(l, c2):
            for ci in range(VECS):
                bvec = bias_v[l, pl.ds(ci * LANES, LANES)]
                for s in range(SEQ_PER_CHUNK):
                    r = s * L + l
                    sl = pl.ds(ci * LANES, LANES)
                    obuf[r, sl] = buf[r, sl] * SCALE + bvec
            return c2

        lax.fori_loop(0, L, pos_body, 0)

    gather_desc = [None] * NBUF
    scatter_desc = [None] * NBUF
    prep(0)
    gather_desc[0] = start_gather(0)
    for g in range(NUM_CHUNKS):
        p = g % NBUF
        if g + 1 < NUM_CHUNKS:
            q = (g + 1) % NBUF
            prep(g + 1)
            gather_desc[q] = start_gather(g + 1)
        gather_desc[p].wait()
        if scatter_desc[p] is not None:
            scatter_desc[p].wait()
        compute(p)
        scatter_desc[p] = pltpu.async_copy(
            obufs[p], out_hbm.at[pl.ds(base + g * CHUNK, CHUNK)], ssems[p])
    for p in range(NBUF):
        if scatter_desc[p] is not None:
            scatter_desc[p].wait()


def kernel(word_ids, word_table, pos_table):
    bias = jnp.asarray(_PE_L) + pos_table[:L]
    ids = word_ids.reshape(ROWS)
    table_pad = jnp.pad(word_table, ((0, 0), (0, PAD_COLS - DIM)))
    out = _embed_sc(ids, table_pad, bias)
    return out.reshape(B, L, DIM)


# speedup vs baseline: 1.1021x; 1.1021x over previous
import functools
import math

import jax
import jax.numpy as jnp
import numpy as np
from jax import lax
from jax.experimental import pallas as pl
from jax.experimental.pallas import tpu as pltpu
from jax.experimental.pallas import tpu_sc as plsc

VOCAB = 1000000
DIM = 64
B = 4096
L = 50
MAX_LEN = 5000

NUM_CORES = 2
NUM_SUBCORES = 16
NUM_WORKERS = NUM_CORES * NUM_SUBCORES
LANES = 16
VECS = DIM // LANES

PAD_COLS = 2 * DIM

ROWS = B * L
ROWS_PER_WORKER = ROWS // NUM_WORKERS
SEQ_PER_CHUNK = 4
CHUNK = SEQ_PER_CHUNK * L
NUM_CHUNKS = ROWS_PER_WORKER // CHUNK
NBUF = 2

SCALE = math.sqrt(DIM)


def _build_pe(dim: int, max_len: int) -> np.ndarray:
    position = np.arange(max_len, dtype=np.float32)[:, None]
    div_term = np.exp(
        np.arange(0, dim, 2, dtype=np.float32) * -(math.log(10000.0) / dim)
    )[None, :]
    pe = np.zeros((max_len, dim), dtype=np.float32)
    pe[:, 0::2] = np.sin(position * div_term)
    pe[:, 1::2] = np.cos(position * div_term)
    return pe


_PE_L = _build_pe(DIM, MAX_LEN)[:L]


_MESH = plsc.VectorSubcoreMesh(
    core_axis_name="c", subcore_axis_name="s",
    num_cores=NUM_CORES, num_subcores=NUM_SUBCORES,
)


@functools.partial(
    pl.kernel,
    out_type=jax.ShapeDtypeStruct((B, L, DIM), jnp.float32),
    mesh=_MESH,
    scratch_types=[
        pltpu.VMEM((ROWS_PER_WORKER,), jnp.int32),
        pltpu.VMEM((L, DIM), jnp.float32),
        [pltpu.VMEM((CHUNK, PAD_COLS), jnp.float32) for _ in range(NBUF)],
        [pltpu.VMEM((SEQ_PER_CHUNK, L, DIM), jnp.float32)
         for _ in range(NBUF)],
        [pltpu.VMEM((CHUNK,), jnp.int32) for _ in range(NBUF)],
        [pltpu.SemaphoreType.DMA for _ in range(NBUF)],
        [pltpu.SemaphoreType.DMA for _ in range(NBUF)],
    ],
    compiler_params=pltpu.CompilerParams(use_tc_tiling_on_sc=True),
)
def _embed_sc(ids_hbm, table_hbm, bias_hbm, out_hbm, idx_v, bias_v, bufs,
              obufs, cidxs, gsems, ssems):
    wid = lax.axis_index("s") * NUM_CORES + lax.axis_index("c")
    base = wid * ROWS_PER_WORKER

    pltpu.sync_copy(bias_hbm, bias_v)
    pltpu.sync_copy(ids_hbm.at[pl.ds(base, ROWS_PER_WORKER)], idx_v)

    def prep(g):
        q = g % NBUF
        starts = list(range(0, CHUNK - LANES + 1, LANES))
        if starts[-1] + LANES < CHUNK:
            starts.append(CHUNK - LANES)
        for j in starts:
            cidxs[q][pl.ds(j, LANES)] = idx_v[pl.ds(g * CHUNK + j, LANES)]

    def start_gather(g):
        q = g % NBUF
        return pltpu.async_copy(table_hbm.at[cidxs[q]], bufs[q], gsems[q])

    def compute(p):
        buf, obuf = bufs[p], obufs[p]

        def pos_body(l, c2):
            for ci in range(VECS):
                bvec = bias_v[l, pl.ds(ci * LANES, LANES)]
                for s in range(SEQ_PER_CHUNK):
                    r = s * L + l
                    sl = pl.ds(ci * LANES, LANES)
                    obuf[s, l, sl] = buf[r, sl] * SCALE + bvec
            return c2

        lax.fori_loop(0, L, pos_body, 0)

    gather_desc = [None] * NBUF
    scatter_desc = [None] * NBUF
    prep(0)
    gather_desc[0] = start_gather(0)
    for g in range(NUM_CHUNKS):
        p = g % NBUF
        if g + 1 < NUM_CHUNKS:
            q = (g + 1) % NBUF
            prep(g + 1)
            gather_desc[q] = start_gather(g + 1)
        gather_desc[p].wait()
        if scatter_desc[p] is not None:
            scatter_desc[p].wait()
        compute(p)
        b0 = (base + g * CHUNK) // L
        scatter_desc[p] = pltpu.async_copy(
            obufs[p], out_hbm.at[pl.ds(b0, SEQ_PER_CHUNK)], ssems[p])
    for p in range(NBUF):
        if scatter_desc[p] is not None:
            scatter_desc[p].wait()


def kernel(word_ids, word_table, pos_table):
    bias = jnp.asarray(_PE_L) + pos_table[:L]
    ids = word_ids.reshape(ROWS)
    table_pad = jnp.pad(word_table, ((0, 0), (0, PAD_COLS - DIM)))
    return _embed_sc(ids, table_pad, bias)
